# baseline (speedup 1.0000x reference)
import jax
import jax.numpy as jnp
from jax.experimental import pallas as pl
from jax.experimental.pallas import tpu as pltpu

_VOCAB = 1000000
_DIM = 64
_CTX = 200
_BLK = 32768


def _cbow_body(idx_ref, emb_hbm, w_ref, b_ref, out_ref, rows_ref, s_ref, sem):
    i = pl.program_id(0)

    @pl.when(i == 0)
    def _gather_and_sum():
        def _issue(j, carry):
            idx = idx_ref[j]
            pltpu.make_async_copy(
                emb_hbm.at[pl.ds(idx, 1), :],
                rows_ref.at[pl.ds(j, 1), :],
                sem,
            ).start()
            return carry

        jax.lax.fori_loop(0, _CTX, _issue, 0)

        def _wait(j, carry):
            pltpu.make_async_copy(
                emb_hbm.at[pl.ds(0, 1), :],
                rows_ref.at[pl.ds(0, 1), :],
                sem,
            ).wait()
            return carry

        jax.lax.fori_loop(0, _CTX, _wait, 0)
        s_ref[...] = jnp.sum(rows_ref[...], axis=0, keepdims=True)

    out_ref[...] = (
        jnp.dot(s_ref[...], w_ref[...], preferred_element_type=jnp.float32)
        + b_ref[...]
    )


def kernel(context_idxs, emb_table, W, b):
    b2 = b.reshape(1, _VOCAB)
    grid = (pl.cdiv(_VOCAB, _BLK),)
    out = pl.pallas_call(
        _cbow_body,
        grid=grid,
        in_specs=[
            pl.BlockSpec(memory_space=pltpu.MemorySpace.SMEM),
            pl.BlockSpec(memory_space=pltpu.MemorySpace.HBM),
            pl.BlockSpec((_DIM, _BLK), lambda i: (0, i)),
            pl.BlockSpec((1, _BLK), lambda i: (0, i)),
        ],
        out_specs=pl.BlockSpec((1, _BLK), lambda i: (0, i)),
        out_shape=jax.ShapeDtypeStruct((1, _VOCAB), jnp.float32),
        scratch_shapes=[
            pltpu.VMEM((_CTX, _DIM), jnp.float32),
            pltpu.VMEM((1, _DIM), jnp.float32),
            pltpu.SemaphoreType.DMA,
        ],
    )(context_idxs, emb_table, W, b2)
    return out
